# Initial kernel scaffold; baseline (speedup 1.0000x reference)
import jax
import jax.numpy as jnp
from jax.experimental import pallas as pl


def kernel(x, edge_index, W_self0, W_neigh0, b0, W_self1, W_neigh1, b1, W_self2, W_neigh2, b2):
    raise NotImplementedError("write your pallas kernel here")


# speedup vs baseline: 6.4148x; 6.4148x over previous
import functools

import jax
import jax.numpy as jnp
from jax import lax
from jax.experimental import pallas as pl
from jax.experimental.pallas import tpu as pltpu
from jax.experimental.pallas import tpu_sc as plsc

_B = 128
_NSUB = 16
_NCORE = 2


@functools.partial(jax.jit,
                   static_argnames=("w", "nb", "n_acc", "with_deg", "tc_tiling"))
def _segsum_sc(table, srcs, dsts, zeros_w, zeros16, ones16, *, w, nb, n_acc,
               with_deg, tc_tiling=False):
    npb = nb // _NSUB
    rows_per = n_acc // _NSUB

    out_type = [jax.ShapeDtypeStruct((_NCORE, n_acc, w), jnp.float32)]
    scratch = [
        pltpu.VMEM((_B,), jnp.int32),
        pltpu.VMEM((_B,), jnp.int32),
        pltpu.VMEM((_B, w), jnp.float32),
        pltpu.VMEM_SHARED((n_acc, w), jnp.float32),
        pltpu.SemaphoreType.DMA,
    ]
    if with_deg:
        out_type.append(jax.ShapeDtypeStruct((_NCORE, n_acc, 16), jnp.float32))
        scratch += [
            pltpu.VMEM((_B, 16), jnp.float32),
            pltpu.VMEM_SHARED((n_acc, 16), jnp.float32),
        ]

    mesh = plsc.VectorSubcoreMesh(core_axis_name="c", subcore_axis_name="s",
                                  num_cores=_NCORE, num_subcores=_NSUB)

    def body(table_ref, srcs_ref, dsts_ref, zw_ref, z16_ref, o16_ref, *rest):
        if with_deg:
            (out, degout, idx_s, idx_d, rows, acc, sem, ones_v, deg_acc) = rest
        else:
            (out, idx_s, idx_d, rows, acc, sem) = rest
        c = lax.axis_index("c")
        s = lax.axis_index("s")
        r0 = s * rows_per

        pltpu.sync_copy(zw_ref.at[pl.ds(r0, rows_per)],
                        acc.at[pl.ds(r0, rows_per)])
        if with_deg:
            pltpu.sync_copy(z16_ref.at[pl.ds(r0, rows_per)],
                            deg_acc.at[pl.ds(r0, rows_per)])
            pltpu.sync_copy(o16_ref, ones_v)
        plsc.subcore_barrier()

        def step(i, carry):
            b = s * npb + i
            pltpu.sync_copy(srcs_ref.at[c, b], idx_s)
            pltpu.sync_copy(dsts_ref.at[c, b], idx_d)
            pltpu.async_copy(table_ref.at[idx_s], rows, sem).wait()
            pltpu.sync_copy(rows, acc.at[idx_d], add=True)
            if with_deg:
                pltpu.sync_copy(ones_v, deg_acc.at[idx_d], add=True)
            return carry

        lax.fori_loop(0, npb, step, 0)
        plsc.subcore_barrier()
        pltpu.sync_copy(acc.at[pl.ds(r0, rows_per)],
                        out.at[c, pl.ds(r0, rows_per)])
        if with_deg:
            pltpu.sync_copy(deg_acc.at[pl.ds(r0, rows_per)],
                            degout.at[c, pl.ds(r0, rows_per)])

    kern = pl.kernel(
        body, out_type=out_type, mesh=mesh, scratch_types=scratch,
        compiler_params=pltpu.CompilerParams(use_tc_tiling_on_sc=tc_tiling))
    return kern(table, srcs, dsts, zeros_w, zeros16, ones16)


_R = 2000


def _dinv_from(degp_blk):
    deg = degp_blk[0, :, 0:1] + degp_blk[1, :, 0:1]
    return 1.0 / jnp.maximum(deg, 1.0)


def _tc_layer0(x, aggp, degp, w_self, w_neigh, b, n):
    d_in = x.shape[1]
    d_h = w_self.shape[1]

    def body(x_ref, aggp_ref, degp_ref, ws_ref, wn_ref, b_ref, out_ref):
        dinv = _dinv_from(degp_ref)
        hn = (aggp_ref[0] + aggp_ref[1]) * dinv
        h = (jnp.dot(x_ref[...], ws_ref[...],
                     preferred_element_type=jnp.float32)
             + jnp.dot(hn, wn_ref[...], preferred_element_type=jnp.float32)
             + b_ref[...])
        h = jnp.maximum(h, 0.0)
        out_ref[0, :, :] = h[:, : d_h // 2]
        out_ref[1, :, :] = h[:, d_h // 2:]

    grid = (n // _R,)
    return pl.pallas_call(
        body,
        grid=grid,
        in_specs=[
            pl.BlockSpec((_R, d_in), lambda i: (i, 0)),
            pl.BlockSpec((2, _R, d_in), lambda i: (0, i, 0)),
            pl.BlockSpec((2, _R, 16), lambda i: (0, i, 0)),
            pl.BlockSpec((d_in, d_h), lambda i: (0, 0)),
            pl.BlockSpec((d_in, d_h), lambda i: (0, 0)),
            pl.BlockSpec((1, d_h), lambda i: (0, 0)),
        ],
        out_specs=pl.BlockSpec((2, _R, d_h // 2), lambda i: (0, i, 0)),
        out_shape=jax.ShapeDtypeStruct((2, n, d_h // 2), jnp.float32),
    )(x, aggp, degp, w_self, w_neigh, b.reshape(1, -1))


def _tc_layer1(h1b, agg1, degp, w_self, w_neigh, b, w_self2, w_neigh2, n):
    d_h = w_self.shape[0]
    d_o = w_self2.shape[1]

    def body(h1b_ref, agg1_ref, degp_ref, ws_ref, wn_ref, b_ref, ws2_ref,
             wn2_ref, s2_ref, p2_ref):
        dinv = _dinv_from(degp_ref)
        h1 = jnp.concatenate([h1b_ref[0], h1b_ref[1]], axis=1)
        agg = jnp.concatenate([agg1_ref[0], agg1_ref[1]], axis=1)
        hn = agg * dinv
        h2 = (jnp.dot(h1, ws_ref[...], preferred_element_type=jnp.float32)
              + jnp.dot(hn, wn_ref[...], preferred_element_type=jnp.float32)
              + b_ref[...])
        h2 = jnp.maximum(h2, 0.0)
        s2_ref[...] = jnp.dot(h2, ws2_ref[...],
                              preferred_element_type=jnp.float32)
        p2_ref[...] = jnp.dot(h2, wn2_ref[...],
                              preferred_element_type=jnp.float32)

    grid = (n // _R,)
    return pl.pallas_call(
        body,
        grid=grid,
        in_specs=[
            pl.BlockSpec((2, _R, d_h // 2), lambda i: (0, i, 0)),
            pl.BlockSpec((2, _R, d_h // 2), lambda i: (0, i, 0)),
            pl.BlockSpec((2, _R, 16), lambda i: (0, i, 0)),
            pl.BlockSpec((d_h, d_h), lambda i: (0, 0)),
            pl.BlockSpec((d_h, d_h), lambda i: (0, 0)),
            pl.BlockSpec((1, d_h), lambda i: (0, 0)),
            pl.BlockSpec((d_h, d_o), lambda i: (0, 0)),
            pl.BlockSpec((d_h, d_o), lambda i: (0, 0)),
        ],
        out_specs=[
            pl.BlockSpec((_R, d_o), lambda i: (i, 0)),
            pl.BlockSpec((_R, d_o), lambda i: (i, 0)),
        ],
        out_shape=[
            jax.ShapeDtypeStruct((n, d_o), jnp.float32),
            jax.ShapeDtypeStruct((n, d_o), jnp.float32),
        ],
    )(h1b, agg1, degp, w_self, w_neigh, b.reshape(1, -1), w_self2, w_neigh2)


def _tc_layer2(s2, aggp, degp, b, n):
    d_o = s2.shape[1]

    def body(s2_ref, aggp_ref, degp_ref, b_ref, out_ref):
        dinv = _dinv_from(degp_ref)
        agg = (aggp_ref[0] + aggp_ref[1]) * dinv
        out_ref[...] = s2_ref[...] + agg + b_ref[...]

    grid = (n // _R,)
    return pl.pallas_call(
        body,
        grid=grid,
        in_specs=[
            pl.BlockSpec((_R, d_o), lambda i: (i, 0)),
            pl.BlockSpec((2, _R, d_o), lambda i: (0, i, 0)),
            pl.BlockSpec((2, _R, 16), lambda i: (0, i, 0)),
            pl.BlockSpec((1, d_o), lambda i: (0, 0)),
        ],
        out_specs=pl.BlockSpec((_R, d_o), lambda i: (i, 0)),
        out_shape=jax.ShapeDtypeStruct((n, d_o), jnp.float32),
    )(s2, aggp, degp, b.reshape(1, -1))


def _ceil_to(x, m):
    return -(-x // m) * m


def kernel(x, edge_index, W_self0, W_neigh0, b0, W_self1, W_neigh1, b1,
           W_self2, W_neigh2, b2):
    n, d_in = x.shape
    e = edge_index.shape[1]
    d_h = W_self1.shape[0]
    d_out = W_self2.shape[1]
    d_op = _ceil_to(d_out, 16)
    n_acc = _ceil_to(n + 16, _NSUB * 8)

    src = edge_index[0]
    dst = edge_index[1]

    nb0 = _ceil_to(-(-e // 2 // _B), _NSUB)
    pad0 = 2 * nb0 * _B - e
    j0 = jnp.arange(pad0, dtype=jnp.int32)
    src0 = jnp.concatenate([src, j0 % n]).reshape(2, nb0, _B)
    dst0 = jnp.concatenate([dst, n + (j0 % 16)]).reshape(2, nb0, _B)

    nb1 = _ceil_to(-(-e // _B), _NSUB)
    pad1 = nb1 * _B - e
    j1 = jnp.arange(pad1, dtype=jnp.int32)
    s1 = jnp.concatenate([src, j1 % n])
    d1 = jnp.concatenate([dst, n + (j1 % 16)])
    src1 = jnp.stack([s1, s1 + n]).reshape(2, nb1, _B)
    dst1 = jnp.stack([d1, d1]).reshape(2, nb1, _B)

    zeros128 = jnp.zeros((n_acc, d_h // 2), jnp.float32)
    zeros48 = jnp.zeros((n_acc, d_op), jnp.float32)
    zeros16 = jnp.zeros((n_acc, 16), jnp.float32)
    ones16 = jnp.ones((_B, 16), jnp.float32)

    agg0p, degp = _segsum_sc(x, src0, dst0, zeros128, zeros16, ones16,
                             w=d_in, nb=nb0, n_acc=n_acc, with_deg=True,
                             tc_tiling=False)
    h1b = _tc_layer0(x, agg0p, degp, W_self0, W_neigh0, b0, n)

    table1 = h1b.reshape(2 * n, d_h // 2)
    (agg1,) = _segsum_sc(table1, src1, dst1, zeros128, zeros16, ones16,
                         w=d_h // 2, nb=nb1, n_acc=n_acc, with_deg=False,
                         tc_tiling=False)

    ws2p = jnp.pad(W_self2, ((0, 0), (0, d_op - d_out)))
    wn2p = jnp.pad(W_neigh2, ((0, 0), (0, d_op - d_out)))
    b2p = jnp.pad(b2, (0, d_op - d_out))
    s2, p2 = _tc_layer1(h1b, agg1, degp, W_self1, W_neigh1, b1, ws2p, wn2p, n)

    (aggp2,) = _segsum_sc(p2, src0, dst0, zeros48, zeros16, ones16,
                          w=d_op, nb=nb0, n_acc=n_acc, with_deg=False,
                          tc_tiling=False)
    out = _tc_layer2(s2, aggp2, degp, b2p, n)
    return out[:, :d_out]


# speedup vs baseline: 13.5330x; 2.1096x over previous
import functools

import jax
import jax.numpy as jnp
from jax import lax
from jax.experimental import pallas as pl
from jax.experimental.pallas import tpu as pltpu
from jax.experimental.pallas import tpu_sc as plsc

_B = 128
_CH = 8
_NSUB = 16
_NCORE = 2


@functools.partial(jax.jit,
                   static_argnames=("w", "nb", "n_acc", "with_deg", "tc_tiling"))
def _segsum_sc(table, srcs, dsts, zeros_w, zeros16, ones16, *, w, nb, n_acc,
               with_deg, tc_tiling=False):
    npb = nb // _NSUB
    nch = npb // _CH
    assert npb % (2 * _CH) == 0, (npb, _CH)
    rows_per = n_acc // _NSUB

    out_type = [jax.ShapeDtypeStruct((_NCORE, n_acc, w), jnp.float32)]
    scratch = [
        pltpu.VMEM((_CH, _B), jnp.int32),
        pltpu.VMEM((_CH, _B), jnp.int32),
        pltpu.VMEM((_CH, _B), jnp.int32),
        pltpu.VMEM((_CH, _B), jnp.int32),
        pltpu.VMEM((_B, w), jnp.float32),
        pltpu.VMEM((_B, w), jnp.float32),
        pltpu.VMEM_SHARED((n_acc, w), jnp.float32),
        pltpu.SemaphoreType.DMA,
        pltpu.SemaphoreType.DMA,
        pltpu.SemaphoreType.DMA,
        pltpu.SemaphoreType.DMA,
    ]
    if with_deg:
        out_type.append(jax.ShapeDtypeStruct((_NCORE, n_acc, 16), jnp.float32))
        scratch += [
            pltpu.VMEM((_B, 16), jnp.float32),
            pltpu.VMEM_SHARED((n_acc, 16), jnp.float32),
        ]

    mesh = plsc.VectorSubcoreMesh(core_axis_name="c", subcore_axis_name="s",
                                  num_cores=_NCORE, num_subcores=_NSUB)

    def body(table_ref, srcs_ref, dsts_ref, zw_ref, z16_ref, o16_ref, *rest):
        if with_deg:
            (out, degout, is0, is1, id0, id1, rows0, rows1, acc,
             semr0, semr1, semi0, semi1, ones_v, deg_acc) = rest
        else:
            (out, is0, is1, id0, id1, rows0, rows1, acc,
             semr0, semr1, semi0, semi1) = rest
        isb = (is0, is1)
        idb = (id0, id1)
        rows = (rows0, rows1)
        semr = (semr0, semr1)
        semi = (semi0, semi1)
        c = lax.axis_index("c")
        s = lax.axis_index("s")
        r0 = s * rows_per
        b0 = s * npb

        def idx_copy(cc, p):
            pltpu.async_copy(srcs_ref.at[c, pl.ds(b0 + cc * _CH, _CH)],
                             isb[p], semi[p])
            return pltpu.async_copy(dsts_ref.at[c, pl.ds(b0 + cc * _CH, _CH)],
                                    idb[p], semi[p])

        def idx_wait(p):
            pltpu.make_async_copy(srcs_ref.at[c, pl.ds(b0, _CH)], isb[p],
                                  semi[p]).wait()
            pltpu.make_async_copy(dsts_ref.at[c, pl.ds(b0, _CH)], idb[p],
                                  semi[p]).wait()

        def gath(p, k, kb):
            return pltpu.async_copy(table_ref.at[isb[p].at[k]], rows[kb],
                                    semr[kb])

        def scat(p, k, kb):
            pltpu.sync_copy(rows[kb], acc.at[idb[p].at[k]], add=True)
            if with_deg:
                pltpu.sync_copy(ones_v, deg_acc.at[idb[p].at[k]], add=True)

        pltpu.sync_copy(zw_ref.at[pl.ds(r0, rows_per)],
                        acc.at[pl.ds(r0, rows_per)])
        if with_deg:
            pltpu.sync_copy(z16_ref.at[pl.ds(r0, rows_per)],
                            deg_acc.at[pl.ds(r0, rows_per)])
            pltpu.sync_copy(o16_ref, ones_v)
        idx_copy(0, 0)
        idx_wait(0)
        idx_copy(1, 1)
        plsc.subcore_barrier()
        gath(0, 0, 0)

        def chunk_body(cc, p):
            for k in range(_CH):
                kb = k % 2
                if k < _CH - 1:
                    gath(p, k + 1, 1 - kb)
                else:
                    @pl.when(cc < nch - 1)
                    def _():
                        idx_wait(1 - p)
                        gath(1 - p, 0, 1 - kb)
                pltpu.make_async_copy(table_ref.at[isb[p].at[k]], rows[kb],
                                      semr[kb]).wait()
                scat(p, k, kb)

            @pl.when(cc < nch - 2)
            def _():
                idx_copy(cc + 2, p)

        def pair(j, carry):
            chunk_body(2 * j, 0)
            chunk_body(2 * j + 1, 1)
            return carry

        lax.fori_loop(0, nch // 2, pair, 0)
        plsc.subcore_barrier()
        pltpu.sync_copy(acc.at[pl.ds(r0, rows_per)],
                        out.at[c, pl.ds(r0, rows_per)])
        if with_deg:
            pltpu.sync_copy(deg_acc.at[pl.ds(r0, rows_per)],
                            degout.at[c, pl.ds(r0, rows_per)])

    kern = pl.kernel(
        body, out_type=out_type, mesh=mesh, scratch_types=scratch,
        compiler_params=pltpu.CompilerParams(use_tc_tiling_on_sc=tc_tiling))
    return kern(table, srcs, dsts, zeros_w, zeros16, ones16)


_R = 2000


def _dinv_from(degp_blk):
    deg = degp_blk[0, :, 0:1] + degp_blk[1, :, 0:1]
    return 1.0 / jnp.maximum(deg, 1.0)


def _tc_layer0(x, aggp, degp, w_self, w_neigh, b, n):
    d_in = x.shape[1]
    d_h = w_self.shape[1]

    def body(x_ref, aggp_ref, degp_ref, ws_ref, wn_ref, b_ref, out_ref):
        dinv = _dinv_from(degp_ref)
        hn = (aggp_ref[0] + aggp_ref[1]) * dinv
        h = (jnp.dot(x_ref[...], ws_ref[...],
                     preferred_element_type=jnp.float32)
             + jnp.dot(hn, wn_ref[...], preferred_element_type=jnp.float32)
             + b_ref[...])
        h = jnp.maximum(h, 0.0)
        out_ref[0, :, :] = h[:, : d_h // 2]
        out_ref[1, :, :] = h[:, d_h // 2:]

    grid = (n // _R,)
    return pl.pallas_call(
        body,
        grid=grid,
        in_specs=[
            pl.BlockSpec((_R, d_in), lambda i: (i, 0)),
            pl.BlockSpec((2, _R, d_in), lambda i: (0, i, 0)),
            pl.BlockSpec((2, _R, 16), lambda i: (0, i, 0)),
            pl.BlockSpec((d_in, d_h), lambda i: (0, 0)),
            pl.BlockSpec((d_in, d_h), lambda i: (0, 0)),
            pl.BlockSpec((1, d_h), lambda i: (0, 0)),
        ],
        out_specs=pl.BlockSpec((2, _R, d_h // 2), lambda i: (0, i, 0)),
        out_shape=jax.ShapeDtypeStruct((2, n, d_h // 2), jnp.float32),
    )(x, aggp, degp, w_self, w_neigh, b.reshape(1, -1))


def _tc_layer1(h1b, agg1, degp, w_self, w_neigh, b, w_self2, w_neigh2, n):
    d_h = w_self.shape[0]
    d_o = w_self2.shape[1]

    def body(h1b_ref, agg1_ref, degp_ref, ws_ref, wn_ref, b_ref, ws2_ref,
             wn2_ref, s2_ref, p2_ref):
        dinv = _dinv_from(degp_ref)
        h1 = jnp.concatenate([h1b_ref[0], h1b_ref[1]], axis=1)
        agg = jnp.concatenate([agg1_ref[0], agg1_ref[1]], axis=1)
        hn = agg * dinv
        h2 = (jnp.dot(h1, ws_ref[...], preferred_element_type=jnp.float32)
              + jnp.dot(hn, wn_ref[...], preferred_element_type=jnp.float32)
              + b_ref[...])
        h2 = jnp.maximum(h2, 0.0)
        s2_ref[...] = jnp.dot(h2, ws2_ref[...],
                              preferred_element_type=jnp.float32)
        p2_ref[...] = jnp.dot(h2, wn2_ref[...],
                              preferred_element_type=jnp.float32)

    grid = (n // _R,)
    return pl.pallas_call(
        body,
        grid=grid,
        in_specs=[
            pl.BlockSpec((2, _R, d_h // 2), lambda i: (0, i, 0)),
            pl.BlockSpec((2, _R, d_h // 2), lambda i: (0, i, 0)),
            pl.BlockSpec((2, _R, 16), lambda i: (0, i, 0)),
            pl.BlockSpec((d_h, d_h), lambda i: (0, 0)),
            pl.BlockSpec((d_h, d_h), lambda i: (0, 0)),
            pl.BlockSpec((1, d_h), lambda i: (0, 0)),
            pl.BlockSpec((d_h, d_o), lambda i: (0, 0)),
            pl.BlockSpec((d_h, d_o), lambda i: (0, 0)),
        ],
        out_specs=[
            pl.BlockSpec((_R, d_o), lambda i: (i, 0)),
            pl.BlockSpec((_R, d_o), lambda i: (i, 0)),
        ],
        out_shape=[
            jax.ShapeDtypeStruct((n, d_o), jnp.float32),
            jax.ShapeDtypeStruct((n, d_o), jnp.float32),
        ],
    )(h1b, agg1, degp, w_self, w_neigh, b.reshape(1, -1), w_self2, w_neigh2)


def _tc_layer2(s2, aggp, degp, b, n):
    d_o = s2.shape[1]

    def body(s2_ref, aggp_ref, degp_ref, b_ref, out_ref):
        dinv = _dinv_from(degp_ref)
        agg = (aggp_ref[0] + aggp_ref[1]) * dinv
        out_ref[...] = s2_ref[...] + agg + b_ref[...]

    grid = (n // _R,)
    return pl.pallas_call(
        body,
        grid=grid,
        in_specs=[
            pl.BlockSpec((_R, d_o), lambda i: (i, 0)),
            pl.BlockSpec((2, _R, d_o), lambda i: (0, i, 0)),
            pl.BlockSpec((2, _R, 16), lambda i: (0, i, 0)),
            pl.BlockSpec((1, d_o), lambda i: (0, 0)),
        ],
        out_specs=pl.BlockSpec((_R, d_o), lambda i: (i, 0)),
        out_shape=jax.ShapeDtypeStruct((n, d_o), jnp.float32),
    )(s2, aggp, degp, b.reshape(1, -1))


def _ceil_to(x, m):
    return -(-x // m) * m


def kernel(x, edge_index, W_self0, W_neigh0, b0, W_self1, W_neigh1, b1,
           W_self2, W_neigh2, b2):
    n, d_in = x.shape
    e = edge_index.shape[1]
    d_h = W_self1.shape[0]
    d_out = W_self2.shape[1]
    d_op = _ceil_to(d_out, 16)
    n_acc = _ceil_to(n + 16, _NSUB * 8)

    src = edge_index[0]
    dst = edge_index[1]

    nb0 = _ceil_to(-(-e // 2 // _B), 2 * _CH * _NSUB)
    pad0 = 2 * nb0 * _B - e
    j0 = jnp.arange(pad0, dtype=jnp.int32)
    src0 = jnp.concatenate([src, j0 % n]).reshape(2, nb0, _B)
    dst0 = jnp.concatenate([dst, n + (j0 % 16)]).reshape(2, nb0, _B)

    nb1 = _ceil_to(-(-e // _B), 2 * _CH * _NSUB)
    pad1 = nb1 * _B - e
    j1 = jnp.arange(pad1, dtype=jnp.int32)
    s1 = jnp.concatenate([src, j1 % n])
    d1 = jnp.concatenate([dst, n + (j1 % 16)])
    src1 = jnp.stack([s1, s1 + n]).reshape(2, nb1, _B)
    dst1 = jnp.stack([d1, d1]).reshape(2, nb1, _B)

    zeros128 = jnp.zeros((n_acc, d_h // 2), jnp.float32)
    zeros48 = jnp.zeros((n_acc, d_op), jnp.float32)
    zeros16 = jnp.zeros((n_acc, 16), jnp.float32)
    ones16 = jnp.ones((_B, 16), jnp.float32)

    agg0p, degp = _segsum_sc(x, src0, dst0, zeros128, zeros16, ones16,
                             w=d_in, nb=nb0, n_acc=n_acc, with_deg=True,
                             tc_tiling=False)
    h1b = _tc_layer0(x, agg0p, degp, W_self0, W_neigh0, b0, n)

    table1 = h1b.reshape(2 * n, d_h // 2)
    (agg1,) = _segsum_sc(table1, src1, dst1, zeros128, zeros16, ones16,
                         w=d_h // 2, nb=nb1, n_acc=n_acc, with_deg=False,
                         tc_tiling=False)

    ws2p = jnp.pad(W_self2, ((0, 0), (0, d_op - d_out)))
    wn2p = jnp.pad(W_neigh2, ((0, 0), (0, d_op - d_out)))
    b2p = jnp.pad(b2, (0, d_op - d_out))
    s2, p2 = _tc_layer1(h1b, agg1, degp, W_self1, W_neigh1, b1, ws2p, wn2p, n)

    (aggp2,) = _segsum_sc(p2, src0, dst0, zeros48, zeros16, ones16,
                          w=d_op, nb=nb0, n_acc=n_acc, with_deg=False,
                          tc_tiling=False)
    out = _tc_layer2(s2, aggp2, degp, b2p, n)
    return out[:, :d_out]


# speedup vs baseline: 13.6012x; 1.0050x over previous
import functools

import jax
import jax.numpy as jnp
from jax import lax
from jax.experimental import pallas as pl
from jax.experimental.pallas import tpu as pltpu
from jax.experimental.pallas import tpu_sc as plsc

_B = 128
_CH = 8
_NSUB = 16
_NCORE = 2


@functools.partial(jax.jit,
                   static_argnames=("w", "nb", "n_acc", "with_deg", "n_off"))
def _segsum_sc(table, srcs, dsts, zeros_w, zeros16, ones16, *, w, nb, n_acc,
               with_deg, n_off=0):
    npb = nb // _NSUB
    nch = npb // _CH
    assert npb % (2 * _CH) == 0, (npb, _CH)
    rows_per = n_acc // _NSUB

    out_type = [jax.ShapeDtypeStruct((_NCORE, n_acc, w), jnp.float32)]
    scratch = [
        pltpu.VMEM((_CH, _B), jnp.int32),
        pltpu.VMEM((_CH, _B), jnp.int32),
        pltpu.VMEM((_CH, _B), jnp.int32),
        pltpu.VMEM((_CH, _B), jnp.int32),
        pltpu.VMEM((_B, w), jnp.float32),
        pltpu.VMEM((_B, w), jnp.float32),
        pltpu.VMEM_SHARED((n_acc, w), jnp.float32),
        pltpu.SemaphoreType.DMA,
        pltpu.SemaphoreType.DMA,
        pltpu.SemaphoreType.DMA,
        pltpu.SemaphoreType.DMA,
    ]
    if with_deg:
        out_type.append(jax.ShapeDtypeStruct((_NCORE, n_acc, 16), jnp.float32))
        scratch += [
            pltpu.VMEM((_B, 16), jnp.float32),
            pltpu.VMEM_SHARED((n_acc, 16), jnp.float32),
        ]

    mesh = plsc.VectorSubcoreMesh(core_axis_name="c", subcore_axis_name="s",
                                  num_cores=_NCORE, num_subcores=_NSUB)

    def body(table_ref, srcs_ref, dsts_ref, zw_ref, z16_ref, o16_ref, *rest):
        if with_deg:
            (out, degout, is0, is1, id0, id1, rows0, rows1, acc,
             semr0, semr1, semi0, semi1, ones_v, deg_acc) = rest
        else:
            (out, is0, is1, id0, id1, rows0, rows1, acc,
             semr0, semr1, semi0, semi1) = rest
        isb = (is0, is1)
        idb = (id0, id1)
        rows = (rows0, rows1)
        semr = (semr0, semr1)
        semi = (semi0, semi1)
        c = lax.axis_index("c")
        s = lax.axis_index("s")
        r0 = s * rows_per
        if n_off:
            b0 = s * npb
            src_off = jnp.broadcast_to((c * n_off).astype(jnp.int32), (16,))
        else:
            b0 = c * nb + s * npb

        def idx_copy(cc, p):
            pltpu.async_copy(srcs_ref.at[pl.ds(b0 + cc * _CH, _CH)],
                             isb[p], semi[p])
            return pltpu.async_copy(dsts_ref.at[pl.ds(b0 + cc * _CH, _CH)],
                                    idb[p], semi[p])

        def idx_wait(p):
            pltpu.make_async_copy(srcs_ref.at[pl.ds(b0, _CH)], isb[p],
                                  semi[p]).wait()
            pltpu.make_async_copy(dsts_ref.at[pl.ds(b0, _CH)], idb[p],
                                  semi[p]).wait()
            if n_off:
                for r in range(_CH):
                    for q in range(_B // 16):
                        sl = (r, pl.ds(q * 16, 16))
                        isb[p][sl] = isb[p][sl] + src_off

        def gath(p, k, kb):
            return pltpu.async_copy(table_ref.at[isb[p].at[k]], rows[kb],
                                    semr[kb])

        def scat(p, k, kb):
            pltpu.sync_copy(rows[kb], acc.at[idb[p].at[k]], add=True)
            if with_deg:
                pltpu.sync_copy(ones_v, deg_acc.at[idb[p].at[k]], add=True)

        pltpu.sync_copy(zw_ref.at[pl.ds(r0, rows_per)],
                        acc.at[pl.ds(r0, rows_per)])
        if with_deg:
            pltpu.sync_copy(z16_ref.at[pl.ds(r0, rows_per)],
                            deg_acc.at[pl.ds(r0, rows_per)])
            pltpu.sync_copy(o16_ref, ones_v)
        idx_copy(0, 0)
        idx_wait(0)
        idx_copy(1, 1)
        plsc.subcore_barrier()
        gath(0, 0, 0)

        def chunk_body(cc, p):
            for k in range(_CH):
                kb = k % 2
                if k < _CH - 1:
                    gath(p, k + 1, 1 - kb)
                else:
                    @pl.when(cc < nch - 1)
                    def _():
                        idx_wait(1 - p)
                        gath(1 - p, 0, 1 - kb)
                pltpu.make_async_copy(table_ref.at[isb[p].at[k]], rows[kb],
                                      semr[kb]).wait()
                scat(p, k, kb)

            @pl.when(cc < nch - 2)
            def _():
                idx_copy(cc + 2, p)

        def pair(j, carry):
            chunk_body(2 * j, 0)
            chunk_body(2 * j + 1, 1)
            return carry

        lax.fori_loop(0, nch // 2, pair, 0)
        plsc.subcore_barrier()
        pltpu.sync_copy(acc.at[pl.ds(r0, rows_per)],
                        out.at[c, pl.ds(r0, rows_per)])
        if with_deg:
            pltpu.sync_copy(deg_acc.at[pl.ds(r0, rows_per)],
                            degout.at[c, pl.ds(r0, rows_per)])

    kern = pl.kernel(
        body, out_type=out_type, mesh=mesh, scratch_types=scratch,
        compiler_params=pltpu.CompilerParams(use_tc_tiling_on_sc=False))
    return kern(table, srcs, dsts, zeros_w, zeros16, ones16)


_R = 2000


def _dinv_from(degp_blk):
    deg = degp_blk[0, :, 0:1] + degp_blk[1, :, 0:1]
    return 1.0 / jnp.maximum(deg, 1.0)


def _tc_layer0(x, aggp, degp, w_self, w_neigh, b, n):
    d_in = x.shape[1]
    d_h = w_self.shape[1]

    def body(x_ref, aggp_ref, degp_ref, ws_ref, wn_ref, b_ref, out_ref):
        dinv = _dinv_from(degp_ref)
        hn = (aggp_ref[0] + aggp_ref[1]) * dinv
        h = (jnp.dot(x_ref[...], ws_ref[...],
                     preferred_element_type=jnp.float32)
             + jnp.dot(hn, wn_ref[...], preferred_element_type=jnp.float32)
             + b_ref[...])
        h = jnp.maximum(h, 0.0)
        out_ref[0, :, :] = h[:, : d_h // 2]
        out_ref[1, :, :] = h[:, d_h // 2:]

    grid = (n // _R,)
    return pl.pallas_call(
        body,
        grid=grid,
        in_specs=[
            pl.BlockSpec((_R, d_in), lambda i: (i, 0)),
            pl.BlockSpec((2, _R, d_in), lambda i: (0, i, 0)),
            pl.BlockSpec((2, _R, 16), lambda i: (0, i, 0)),
            pl.BlockSpec((d_in, d_h), lambda i: (0, 0)),
            pl.BlockSpec((d_in, d_h), lambda i: (0, 0)),
            pl.BlockSpec((1, d_h), lambda i: (0, 0)),
        ],
        out_specs=pl.BlockSpec((2, _R, d_h // 2), lambda i: (0, i, 0)),
        out_shape=jax.ShapeDtypeStruct((2, n, d_h // 2), jnp.float32),
    )(x, aggp, degp, w_self, w_neigh, b.reshape(1, -1))


def _tc_layer1(h1b, agg1, degp, w_self, w_neigh, b, w_self2, w_neigh2, n):
    d_h = w_self.shape[0]
    d_o = w_self2.shape[1]

    def body(h1b_ref, agg1_ref, degp_ref, ws_ref, wn_ref, b_ref, ws2_ref,
             wn2_ref, s2_ref, p2_ref):
        dinv = _dinv_from(degp_ref)
        h1 = jnp.concatenate([h1b_ref[0], h1b_ref[1]], axis=1)
        agg = jnp.concatenate([agg1_ref[0], agg1_ref[1]], axis=1)
        hn = agg * dinv
        h2 = (jnp.dot(h1, ws_ref[...], preferred_element_type=jnp.float32)
              + jnp.dot(hn, wn_ref[...], preferred_element_type=jnp.float32)
              + b_ref[...])
        h2 = jnp.maximum(h2, 0.0)
        s2_ref[...] = jnp.dot(h2, ws2_ref[...],
                              preferred_element_type=jnp.float32)
        p2_ref[...] = jnp.dot(h2, wn2_ref[...],
                              preferred_element_type=jnp.float32)

    grid = (n // _R,)
    return pl.pallas_call(
        body,
        grid=grid,
        in_specs=[
            pl.BlockSpec((2, _R, d_h // 2), lambda i: (0, i, 0)),
            pl.BlockSpec((2, _R, d_h // 2), lambda i: (0, i, 0)),
            pl.BlockSpec((2, _R, 16), lambda i: (0, i, 0)),
            pl.BlockSpec((d_h, d_h), lambda i: (0, 0)),
            pl.BlockSpec((d_h, d_h), lambda i: (0, 0)),
            pl.BlockSpec((1, d_h), lambda i: (0, 0)),
            pl.BlockSpec((d_h, d_o), lambda i: (0, 0)),
            pl.BlockSpec((d_h, d_o), lambda i: (0, 0)),
        ],
        out_specs=[
            pl.BlockSpec((_R, d_o), lambda i: (i, 0)),
            pl.BlockSpec((_R, d_o), lambda i: (i, 0)),
        ],
        out_shape=[
            jax.ShapeDtypeStruct((n, d_o), jnp.float32),
            jax.ShapeDtypeStruct((n, d_o), jnp.float32),
        ],
    )(h1b, agg1, degp, w_self, w_neigh, b.reshape(1, -1), w_self2, w_neigh2)


def _tc_layer2(s2, aggp, degp, b, n):
    d_o = s2.shape[1]

    def body(s2_ref, aggp_ref, degp_ref, b_ref, out_ref):
        dinv = _dinv_from(degp_ref)
        agg = (aggp_ref[0] + aggp_ref[1]) * dinv
        out_ref[...] = s2_ref[...] + agg + b_ref[...]

    grid = (n // _R,)
    return pl.pallas_call(
        body,
        grid=grid,
        in_specs=[
            pl.BlockSpec((_R, d_o), lambda i: (i, 0)),
            pl.BlockSpec((2, _R, d_o), lambda i: (0, i, 0)),
            pl.BlockSpec((2, _R, 16), lambda i: (0, i, 0)),
            pl.BlockSpec((1, d_o), lambda i: (0, 0)),
        ],
        out_specs=pl.BlockSpec((_R, d_o), lambda i: (i, 0)),
        out_shape=jax.ShapeDtypeStruct((n, d_o), jnp.float32),
    )(s2, aggp, degp, b.reshape(1, -1))


def _ceil_to(x, m):
    return -(-x // m) * m


def kernel(x, edge_index, W_self0, W_neigh0, b0, W_self1, W_neigh1, b1,
           W_self2, W_neigh2, b2):
    n, d_in = x.shape
    e = edge_index.shape[1]
    d_h = W_self1.shape[0]
    d_out = W_self2.shape[1]
    d_op = _ceil_to(d_out, 16)
    n_acc = _ceil_to(n + 16, _NSUB * 8)

    src = edge_index[0]
    dst = edge_index[1]

    nb1 = _ceil_to(-(-e // _B), 2 * 2 * _CH * _NSUB)
    nb0 = nb1 // 2
    pad1 = nb1 * _B - e
    j1 = jnp.arange(pad1, dtype=jnp.int32)
    src_p = jnp.concatenate([src, j1 % n]).reshape(nb1, _B)
    dst_p = jnp.concatenate([dst, n + (j1 % 16)]).reshape(nb1, _B)

    zeros128 = jnp.zeros((n_acc, d_h // 2), jnp.float32)
    zeros48 = jnp.zeros((n_acc, d_op), jnp.float32)
    zeros16 = jnp.zeros((n_acc, 16), jnp.float32)
    ones16 = jnp.ones((_B, 16), jnp.float32)

    agg0p, degp = _segsum_sc(x, src_p, dst_p, zeros128, zeros16, ones16,
                             w=d_in, nb=nb0, n_acc=n_acc, with_deg=True)
    h1b = _tc_layer0(x, agg0p, degp, W_self0, W_neigh0, b0, n)

    table1 = h1b.reshape(2 * n, d_h // 2)
    (agg1,) = _segsum_sc(table1, src_p, dst_p, zeros128, zeros16, ones16,
                         w=d_h // 2, nb=nb1, n_acc=n_acc, with_deg=False,
                         n_off=n)

    ws2p = jnp.pad(W_self2, ((0, 0), (0, d_op - d_out)))
    wn2p = jnp.pad(W_neigh2, ((0, 0), (0, d_op - d_out)))
    b2p = jnp.pad(b2, (0, d_op - d_out))
    s2, p2 = _tc_layer1(h1b, agg1, degp, W_self1, W_neigh1, b1, ws2p, wn2p, n)

    (aggp2,) = _segsum_sc(p2, src_p, dst_p, zeros48, zeros16, ones16,
                          w=d_op, nb=nb0, n_acc=n_acc, with_deg=False)
    out = _tc_layer2(s2, aggp2, degp, b2p, n)
    return out[:, :d_out]
